# baseline (speedup 1.0000x reference)
import functools

import jax
import jax.numpy as jnp
from jax import lax
from jax.experimental import pallas as pl
from jax.experimental.pallas import tpu as pltpu
from jax.experimental.pallas import tpu_sc as plsc

_NC = 2
_NS = 16
_NW = _NC * _NS


def _prep_body(x_ref, seg_ref, tok_ref, segw_ref, pos_ref, gam_ref, bet_ref,
               table_ref, idx_ref):
    nv, d = tok_ref.shape
    ns = segw_ref.shape[0]
    npos = pos_ref.shape[0]
    n = nv * ns * npos

    row = lax.broadcasted_iota(jnp.int32, (n, 1), 0)
    ohv = (row // (ns * npos) == lax.broadcasted_iota(jnp.int32, (n, nv), 1))
    ohs = ((row // npos) % ns == lax.broadcasted_iota(jnp.int32, (n, ns), 1))
    ohp = (row % npos == lax.broadcasted_iota(jnp.int32, (n, npos), 1))

    dot = functools.partial(jnp.dot, preferred_element_type=jnp.float32,
                            precision=lax.Precision.HIGHEST)
    emb = (dot(ohv.astype(jnp.float32), tok_ref[...])
           + dot(ohs.astype(jnp.float32), segw_ref[...])
           + dot(ohp.astype(jnp.float32), pos_ref[...]))

    mu = jnp.mean(emb, axis=-1, keepdims=True)
    var = jnp.mean((emb - mu) ** 2, axis=-1, keepdims=True)
    table_ref[...] = ((emb - mu) * lax.rsqrt(var + 1e-5) * gam_ref[...]
                      + bet_ref[...])

    l_iota = lax.broadcasted_iota(jnp.int32, x_ref.shape, 1)
    idx_ref[...] = x_ref[...] * (ns * npos) + seg_ref[...] * npos + l_iota


def _sc_gather(table, idx_w, n_rows, d, chunk, n_chunks):
    mesh = plsc.VectorSubcoreMesh(core_axis_name="c", subcore_axis_name="s")

    @functools.partial(
        pl.kernel,
        out_type=jax.ShapeDtypeStruct((n_rows, d), jnp.float32),
        mesh=mesh,
        scratch_types=[
            pltpu.VMEM((n_chunks, chunk), jnp.int32),
            pltpu.VMEM((chunk, d), jnp.float32),
            pltpu.VMEM((chunk, d), jnp.float32),
            pltpu.SemaphoreType.DMA,
            pltpu.SemaphoreType.DMA,
        ],
    )
    def run(table_hbm, idx_hbm, out_hbm, idx_v, rows0, rows1, gsem0, gsem1):
        wid = lax.axis_index("s") * _NC + lax.axis_index("c")
        base = wid * (n_chunks * chunk)
        pltpu.sync_copy(idx_hbm.at[wid], idx_v)
        bufs = (rows0, rows1)
        gsems = (gsem0, gsem1)

        def gather_start(g, b):
            pltpu.async_copy(table_hbm.at[idx_v.at[g]], bufs[b], gsems[b])

        def gather_wait(g, b):
            pltpu.make_async_copy(table_hbm.at[idx_v.at[g]], bufs[b],
                                  gsems[b]).wait()

        gather_start(0, 0)
        gather_start(1, 1)

        def step(i, carry):
            for b in range(2):
                g = i * 2 + b
                gather_wait(g, b)
                pltpu.sync_copy(bufs[b], out_hbm.at[pl.ds(base + g * chunk,
                                                          chunk)])

                @pl.when(g + 2 < n_chunks)
                def _():
                    gather_start(g + 2, b)
            return carry

        lax.fori_loop(0, n_chunks // 2, step, 0)

    return run(table, idx_w)


def kernel(x, seg, tok_w, seg_w, pos_w, gamma, beta):
    b, l = x.shape
    nv, d = tok_w.shape
    ns = seg_w.shape[0]
    npos = pos_w.shape[0]
    n_rows = b * l

    table, idx = pl.pallas_call(
        _prep_body,
        out_shape=[
            jax.ShapeDtypeStruct((nv * ns * npos, d), jnp.float32),
            jax.ShapeDtypeStruct((b, l), jnp.int32),
        ],
    )(x, seg, tok_w, seg_w, pos_w, gamma.reshape(1, d), beta.reshape(1, d))

    chunk = 16
    n_chunks = n_rows // (_NW * chunk)
    idx_w = idx.T.reshape(_NW, n_chunks, chunk)
    flat = _sc_gather(table, idx_w, n_rows, d, chunk, n_chunks)
    return flat.reshape(l, b, d).transpose(1, 0, 2)


# speedup vs baseline: 1.0373x; 1.0373x over previous
import functools

import jax
import jax.numpy as jnp
from jax import lax
from jax.experimental import pallas as pl
from jax.experimental.pallas import tpu as pltpu
from jax.experimental.pallas import tpu_sc as plsc

_NC = 2
_NS = 16
_NW = _NC * _NS


def _prep_body(x_ref, seg_ref, tok_ref, segw_ref, pos_ref, gam_ref, bet_ref,
               table_ref, idx_ref):
    nv, d = tok_ref.shape
    ns = segw_ref.shape[0]
    npos = pos_ref.shape[0]
    n = nv * ns * npos

    row = lax.broadcasted_iota(jnp.int32, (n, 1), 0)
    ohv = (row // (ns * npos) == lax.broadcasted_iota(jnp.int32, (n, nv), 1))
    ohs = ((row // npos) % ns == lax.broadcasted_iota(jnp.int32, (n, ns), 1))
    ohp = (row % npos == lax.broadcasted_iota(jnp.int32, (n, npos), 1))

    dot = functools.partial(jnp.dot, preferred_element_type=jnp.float32,
                            precision=lax.Precision.HIGHEST)
    emb = (dot(ohv.astype(jnp.float32), tok_ref[...])
           + dot(ohs.astype(jnp.float32), segw_ref[...])
           + dot(ohp.astype(jnp.float32), pos_ref[...]))

    mu = jnp.mean(emb, axis=-1, keepdims=True)
    var = jnp.mean((emb - mu) ** 2, axis=-1, keepdims=True)
    table_ref[...] = ((emb - mu) * lax.rsqrt(var + 1e-5) * gam_ref[...]
                      + bet_ref[...])

    l_iota = lax.broadcasted_iota(jnp.int32, x_ref.shape, 1)
    idx_ref[...] = x_ref[...] * (ns * npos) + seg_ref[...] * npos + l_iota


def _sc_gather(table, idx_w, n_rows, d, chunk, n_chunks):
    mesh = plsc.VectorSubcoreMesh(core_axis_name="c", subcore_axis_name="s")

    @functools.partial(
        pl.kernel,
        out_type=jax.ShapeDtypeStruct((n_rows, d), jnp.float32),
        mesh=mesh,
        scratch_types=[
            pltpu.VMEM((n_chunks, chunk), jnp.int32),
            pltpu.VMEM((chunk, d), jnp.float32),
            pltpu.VMEM((chunk, d), jnp.float32),
            pltpu.SemaphoreType.DMA,
            pltpu.SemaphoreType.DMA,
        ],
    )
    def run(table_hbm, idx_hbm, out_hbm, idx_v, rows0, rows1, gsem0, gsem1):
        wid = lax.axis_index("s") * _NC + lax.axis_index("c")
        base = wid * (n_chunks * chunk)
        pltpu.sync_copy(idx_hbm.at[wid], idx_v)
        bufs = (rows0, rows1)
        gsems = (gsem0, gsem1)

        def gather_start(g, b):
            pltpu.async_copy(table_hbm.at[idx_v.at[g]], bufs[b], gsems[b])

        def gather_wait(g, b):
            pltpu.make_async_copy(table_hbm.at[idx_v.at[g]], bufs[b],
                                  gsems[b]).wait()

        gather_start(0, 0)
        gather_start(1, 1)

        def step(i, carry):
            for b in range(2):
                g = i * 2 + b
                gather_wait(g, b)
                pltpu.sync_copy(bufs[b], out_hbm.at[pl.ds(base + g * chunk,
                                                          chunk)])

                @pl.when(g + 2 < n_chunks)
                def _():
                    gather_start(g + 2, b)
            return carry

        lax.fori_loop(0, n_chunks // 2, step, 0)

    return run(table, idx_w)


def kernel(x, seg, tok_w, seg_w, pos_w, gamma, beta):
    b, l = x.shape
    nv, d = tok_w.shape
    ns = seg_w.shape[0]
    npos = pos_w.shape[0]
    n_rows = b * l

    table, idx = pl.pallas_call(
        _prep_body,
        out_shape=[
            jax.ShapeDtypeStruct((nv * ns * npos, d), jnp.float32),
            jax.ShapeDtypeStruct((b, l), jnp.int32),
        ],
    )(x, seg, tok_w, seg_w, pos_w, gamma.reshape(1, d), beta.reshape(1, d))

    chunk = 32
    n_chunks = n_rows // (_NW * chunk)
    idx_w = idx.T.reshape(_NW, n_chunks, chunk)
    flat = _sc_gather(table, idx_w, n_rows, d, chunk, n_chunks)
    return flat.reshape(l, b, d).transpose(1, 0, 2)
